# baseline (speedup 1.0000x reference)
import numpy as np
import jax
import jax.numpy as jnp
from jax.experimental import pallas as pl
from jax.experimental.pallas import tpu as pltpu

B, N, IN_DIM, HID, HIST, FCST = 32, 256, 8, 32, 24, 24
BN = B * N
WIND_THRESH = 0.5
H3 = 3 * HID


def _gru(gi, gh, h):
    ir, iz, in_ = gi[:, 0:HID], gi[:, HID:2 * HID], gi[:, 2 * HID:H3]
    hr, hz, hn_ = gh[:, 0:HID], gh[:, HID:2 * HID], gh[:, 2 * HID:H3]
    r = jax.nn.sigmoid(ir + hr)
    z = jax.nn.sigmoid(iz + hz)
    n = jnp.tanh(in_ + r * hn_)
    return (1.0 - z) * n + z * h


def _body(featR, pm25R, adj, angles,
          Wihh, Whhh, bihh, bhhh, WfhT, bfh,
          W0T, W1T, bch, Wih, Whh, bih, bhh, WfoT, bfo,
          out, y_scr, g_scr):
    c1 = jnp.cos(angles[:, :])
    c2 = jnp.cos(angles[:, :] - np.float32(np.pi / 2))
    adjm = adj[:, :] > 0.0

    whh_h = Whhh[:, :]
    wr0 = Wihh[0:1, :]
    wr1 = Wihh[1:2, :]
    bihh_v = bihh[:, :]
    bhhh_v = bhhh[:, :]
    wfhT = WfhT[:, :]
    bfh_v = bfh[:, :]

    h = jnp.zeros((BN, HID), jnp.float32)
    xn = jnp.zeros((BN, 1), jnp.float32)

    for t in range(HIST):
        pmcol = pm25R[:, t:t + 1]
        gi = xn * wr0 + pmcol * wr1 + bihh_v
        gh = jnp.dot(h, whh_h, preferred_element_type=jnp.float32) + bhhh_v
        h = _gru(gi, gh, h)
        xn = jnp.sum(h * wfhT, axis=1, keepdims=True) + bfh_v

    w0T = W0T[:, :]
    w1T = W1T[:, :]
    bch_v = bch[:, :]
    wih = Wih[:, :]
    whh = Whh[:, :]
    bih_v = bih[:, :]
    bhh_v = bhh[:, :]
    wfoT = WfoT[:, :]
    bfo_v = bfo[:, :]

    for t in range(FCST):
        c0 = t * IN_DIM
        feat_t = featR[:, c0:c0 + IN_DIM]
        y = xn * w1T[:, 0:1] + jnp.sum(feat_t * w1T[:, 1:], axis=1, keepdims=True)
        t0 = xn * w0T[:, 0:1] + jnp.sum(feat_t * w0T[:, 1:], axis=1, keepdims=True)
        y_scr[:, :] = y

        def bbody(b, carry):
            r0 = b * N
            uv = featR[pl.ds(r0, N), c0:c0 + 2]
            gate = uv[:, 0:1] * c1 + uv[:, 1:2] * c2
            A = jnp.where(jnp.logical_and(gate >= WIND_THRESH, adjm), 1.0, 0.0)
            deg = jnp.sum(A, axis=1, keepdims=True)
            dinv = jnp.where(deg > 0.0, jax.lax.rsqrt(deg), 0.0)
            w = dinv * y_scr[pl.ds(r0, N), :]
            s = jnp.dot(A, w, preferred_element_type=jnp.float32)
            g_scr[pl.ds(r0, N), :] = -dinv * s
            return carry

        jax.lax.fori_loop(0, B, bbody, 0)

        xg = jax.nn.sigmoid(t0 + g_scr[:, :] + bch_v)
        gi = (xn * wih[0:1, :]
              + jnp.dot(feat_t, wih[1:1 + IN_DIM, :], preferred_element_type=jnp.float32)
              + xg * wih[1 + IN_DIM:2 + IN_DIM, :]
              + bih_v)
        gh = jnp.dot(h, whh, preferred_element_type=jnp.float32) + bhh_v
        h = _gru(gi, gh, h)
        xn = jnp.sum(h * wfoT, axis=1, keepdims=True) + bfo_v
        out[:, t:t + 1] = xn


def _call(interpret=False):
    return pl.pallas_call(
        _body,
        out_shape=jax.ShapeDtypeStruct((BN, FCST), jnp.float32),
        scratch_shapes=[
            pltpu.VMEM((BN, 1), jnp.float32),
            pltpu.VMEM((BN, 1), jnp.float32),
        ],
        interpret=interpret,
    )


def kernel(feature, pm25_hist, adj_mat, angles, W_ih_h, W_hh_h, b_ih_h, b_hh_h,
           W_fh, b_fh, W0, W1, b_cheb, W_ih, W_hh, b_ih, b_hh, W_fo, b_fo):
    featR = feature[:, HIST:].transpose(0, 2, 1, 3).reshape(BN, FCST * IN_DIM)
    pm25R = pm25_hist[:, :, :, 0].transpose(0, 2, 1).reshape(BN, HIST)
    out = _call()(
        featR, pm25R, adj_mat, angles,
        W_ih_h, W_hh_h,
        b_ih_h.reshape(1, H3), b_hh_h.reshape(1, H3),
        W_fh.reshape(1, HID), b_fh.reshape(1, 1),
        W0.reshape(1, 1 + IN_DIM), W1.reshape(1, 1 + IN_DIM), b_cheb.reshape(1, 1),
        W_ih, W_hh,
        b_ih.reshape(1, H3), b_hh.reshape(1, H3),
        W_fo.reshape(1, HID), b_fo.reshape(1, 1),
    )
    return out.reshape(B, N, FCST).transpose(0, 2, 1)[..., None]
